# baseline (speedup 1.0000x reference)
import functools

import jax
import jax.numpy as jnp
from jax import lax
from jax.experimental import pallas as pl
from jax.experimental.pallas import tpu as pltpu
from jax.experimental.pallas import tpu_sc as plsc

BATCH = 16384
DIM = 64
ODIM = 2 * DIM
AVOCAB = 100000

_info = plsc.get_sparse_core_info()
_NC = _info.num_cores
_NS = _info.num_subcores
_NW = _NC * _NS
_BPW = BATCH // _NW
_CH = 128
_NCHUNK = _BPW // _CH
_CPP = 2
_PR = _CPP * _CH
_NPASS = _NCHUNK // _CPP


def _build():
    mesh = plsc.VectorSubcoreMesh(core_axis_name="c", subcore_axis_name="s")

    @functools.partial(
        pl.kernel,
        mesh=mesh,
        out_type=jax.ShapeDtypeStruct((BATCH, ODIM), jnp.float32),
        scratch_types=[
            pltpu.VMEM((_NCHUNK, _CH), jnp.int32),
            pltpu.VMEM((_NCHUNK, _CH), jnp.int32),
            pltpu.VMEM((_BPW,), jnp.int32),
            pltpu.VMEM((_PR, ODIM), jnp.float32),
            pltpu.VMEM((_PR, ODIM), jnp.float32),
            pltpu.SemaphoreType.DMA,
        ],
    )
    def k(pidx_hbm, instr_hbm, aoff_hbm, apair_hbm, ipad_hbm, out_hbm,
          pidx_v, iidx_v, aoff_v, a_v, b_v, sem):
        wid = lax.axis_index("s") * _NC + lax.axis_index("c")
        base = wid * _BPW
        row0 = wid * _NCHUNK
        pltpu.sync_copy(pidx_hbm.at[pl.ds(row0, _NCHUNK), :], pidx_v)
        pltpu.sync_copy(instr_hbm.at[pl.ds(row0, _NCHUNK), :], iidx_v)
        pltpu.sync_copy(aoff_hbm.at[pl.ds(base, _BPW)], aoff_v)
        for p in range(_NPASS):
            copies = []
            for j in range(_CPP):
                c = p * _CPP + j
                copies.append(pltpu.async_copy(
                    apair_hbm.at[pidx_v.at[c]],
                    a_v.at[pl.ds(j * _CH, _CH)], sem))
                copies.append(pltpu.async_copy(
                    ipad_hbm.at[iidx_v.at[c]],
                    b_v.at[pl.ds(j * _CH, _CH)], sem))
            for cp in copies:
                cp.wait()

            def mergegrp(g, _):
                offv = aoff_v[pl.ds(p * _PR + g * 16, 16)]
                for i in range(16):
                    r = g * 16 + i
                    off = offv[i]
                    for k16 in range(DIM // 16):
                        b_v[r, pl.ds(k16 * 16, 16)] = (
                            a_v[r, pl.ds(off + k16 * 16, 16)])
                return ()

            lax.fori_loop(0, _PR // 16, mergegrp, ())
            pltpu.sync_copy(b_v, out_hbm.at[pl.ds(base + p * _PR, _PR), :])

    return k


_sc_kernel = _build()


def kernel(adduct, instrument_type, adduct_table, instrument_type_table):
    apair = adduct_table[:AVOCAB].reshape(AVOCAB // 2, ODIM)
    ipad = jnp.pad(instrument_type_table, ((0, 0), (DIM, 0)))
    pidx2 = (adduct >> 1).reshape(_NW * _NCHUNK, _CH)
    aoff = (adduct & 1) * DIM
    instr2 = instrument_type.reshape(_NW * _NCHUNK, _CH)
    return _sc_kernel(pidx2, instr2, aoff, apair, ipad)
